# Initial kernel scaffold; baseline (speedup 1.0000x reference)
import jax
import jax.numpy as jnp
from jax.experimental import pallas as pl


def kernel(logits):
    raise NotImplementedError("write your pallas kernel here")


# speedup vs baseline: 184.2872x; 184.2872x over previous
import jax
import jax.numpy as jnp
from jax.experimental import pallas as pl
from jax.experimental.pallas import tpu as pltpu

_Q = 0.9
_K = 1
_ROWS_PER_BLOCK = 8


def _body(x_ref, t_ref):
    x = x_ref[...]
    m = jnp.max(x, axis=1, keepdims=True)
    e = jnp.exp(x - m)
    z = jnp.sum(e, axis=1, keepdims=True)
    s = e / z
    s1 = s[:, _K:_K + 1]
    col = jax.lax.broadcasted_iota(jnp.int32, x.shape, 1)
    sel = (s > s1) | ((s == s1) & (col <= _K))
    t = jnp.sum(jnp.where(sel, s, 0.0), axis=1, keepdims=True)
    t_ref[...] = jnp.where(t <= _Q, 1.0, 0.0)


def kernel(logits):
    b, v = logits.shape
    r = _ROWS_PER_BLOCK
    out = pl.pallas_call(
        _body,
        grid=(b // r,),
        in_specs=[pl.BlockSpec((r, v), lambda i: (i, 0))],
        out_specs=pl.BlockSpec((r, 1), lambda i: (i, 0)),
        out_shape=jax.ShapeDtypeStruct((b, 1), jnp.float32),
    )(logits)
    preds = out > 0.5
    return preds, ~preds
